# baseline (speedup 1.0000x reference)
import functools

import jax
import jax.numpy as jnp
from jax import lax
from jax.experimental import pallas as pl
from jax.experimental.pallas import tpu as pltpu
from jax.experimental.pallas import tpu_sc as plsc

NC = 1
NS = 16
L = 16
D = 64
PW = 128
NW = NC * NS

CHUNK = 128


def _make_gather(batch):
    bpw = batch // NW
    n_chunks = bpw // CHUNK
    mesh = plsc.VectorSubcoreMesh(core_axis_name="c", subcore_axis_name="s", num_cores=NC)

    @functools.partial(
        pl.kernel,
        mesh=mesh,
        out_type=jax.ShapeDtypeStruct((batch, PW), jnp.float32),
        compiler_params=pltpu.CompilerParams(use_tc_tiling_on_sc=True),
        scratch_types=[
            pltpu.VMEM((bpw,), jnp.int32),
            pltpu.VMEM((CHUNK,), jnp.int32),
            pltpu.VMEM((CHUNK, PW), jnp.float32),
            pltpu.SemaphoreType.DMA,
        ],
    )
    def k(idx_hbm, tbl_hbm, emb_hbm, idx_v, row_v, rows_v, sem):
        wid = lax.axis_index("s") * NC + lax.axis_index("c")
        base = wid * bpw
        pltpu.sync_copy(idx_hbm.at[pl.ds(base, bpw)], idx_v)

        def chunk_body(c, _):
            c0 = c * CHUNK
            for v in range(CHUNK // L):
                row_v[pl.ds(v * L, L)] = idx_v[pl.ds(c0 + v * L, L)] >> 1
            pltpu.async_copy(tbl_hbm.at[row_v], rows_v, sem).wait()
            pltpu.sync_copy(rows_v, emb_hbm.at[pl.ds(base + c0, CHUNK)])
            return _

        lax.fori_loop(0, n_chunks, chunk_body, None)

    return k


def _make_dot(batch):
    bpw = batch // NW
    n_chunks = bpw // CHUNK
    mesh = plsc.VectorSubcoreMesh(core_axis_name="c", subcore_axis_name="s", num_cores=NC)

    @functools.partial(
        pl.kernel,
        mesh=mesh,
        out_type=jax.ShapeDtypeStruct((batch,), jnp.float32),
        compiler_params=pltpu.CompilerParams(use_tc_tiling_on_sc=True),
        scratch_types=[
            pltpu.VMEM((bpw,), jnp.int32),
            pltpu.VMEM((bpw,), jnp.int32),
            pltpu.VMEM((CHUNK, PW), jnp.float32),
            pltpu.VMEM((CHUNK, PW), jnp.float32),
            pltpu.VMEM((bpw,), jnp.float32),
        ],
    )
    def k(user_hbm, item_hbm, uemb_hbm, iemb_hbm, out_hbm,
          uidx_v, iidx_v, urows_v, irows_v, out_v):
        wid = lax.axis_index("s") * NC + lax.axis_index("c")
        base = wid * bpw

        pltpu.sync_copy(user_hbm.at[pl.ds(base, bpw)], uidx_v)
        pltpu.sync_copy(item_hbm.at[pl.ds(base, bpw)], iidx_v)

        lanes = lax.iota(jnp.int32, L)

        def perm_xor(v, s):
            return v.at[lanes ^ s].get(mode="promise_in_bounds")

        def chunk_body(c, _):
            c0 = c * CHUNK
            pltpu.sync_copy(uemb_hbm.at[pl.ds(base + c0, CHUNK)], urows_v)
            pltpu.sync_copy(iemb_hbm.at[pl.ds(base + c0, CHUNK)], irows_v)

            def group(g, _):
                r0 = g * L
                usel = (uidx_v[pl.ds(c0 + r0, L)] & 1) * D
                isel = (iidx_v[pl.ds(c0 + r0, L)] & 1) * D
                accs = []
                for r in range(L):
                    uo = usel[r]
                    io = isel[r]
                    acc = (urows_v[r0 + r, pl.ds(uo, L)] *
                           irows_v[r0 + r, pl.ds(io, L)])
                    for jc in range(1, D // L):
                        acc = acc + (
                            urows_v[r0 + r, pl.ds(uo + jc * L, L)] *
                            irows_v[r0 + r, pl.ds(io + jc * L, L)])
                    accs.append(acc)
                s = 1
                while len(accs) > 1:
                    lo_mask = (lanes & s) == 0
                    nxt = []
                    for i in range(0, len(accs), 2):
                        a, b = accs[i], accs[i + 1]
                        merged = (jnp.where(lo_mask, a, perm_xor(b, s)) +
                                  jnp.where(lo_mask, perm_xor(a, s), b))
                        nxt.append(merged)
                    accs = nxt
                    s *= 2
                out_v[pl.ds(c0 + r0, L)] = accs[0]
                return _

            lax.fori_loop(0, CHUNK // L, group, None)
            return _

        lax.fori_loop(0, n_chunks, chunk_body, None)

        pltpu.sync_copy(out_v, out_hbm.at[pl.ds(base, bpw)])

    return k


def kernel(user, item, user_table, item_table):
    batch = user.shape[0]
    n_rows, dim = user_table.shape
    gather = _make_gather(batch)
    ut2 = user_table.reshape(n_rows * dim // PW, PW)
    it2 = item_table.reshape(n_rows * dim // PW, PW)
    uemb = gather(user, ut2)
    iemb = gather(item, it2)
    return _make_dot(batch)(user, item, uemb, iemb)


# speedup vs baseline: 1.0194x; 1.0194x over previous
import functools

import jax
import jax.numpy as jnp
from jax import lax
from jax.experimental import pallas as pl
from jax.experimental.pallas import tpu as pltpu
from jax.experimental.pallas import tpu_sc as plsc

NC = 2
NS = 16
L = 16
D = 64
PW = 128
NW = NC * NS

CHUNK = 128


def _make_gather(batch):
    bpw = batch // NW
    n_chunks = bpw // CHUNK
    mesh = plsc.VectorSubcoreMesh(core_axis_name="c", subcore_axis_name="s", num_cores=NC)

    @functools.partial(
        pl.kernel,
        mesh=mesh,
        out_type=jax.ShapeDtypeStruct((batch, PW), jnp.float32),
        compiler_params=pltpu.CompilerParams(
            use_tc_tiling_on_sc=True, skip_device_barrier=True),
        scratch_types=[
            pltpu.VMEM((bpw,), jnp.int32),
            pltpu.VMEM((CHUNK,), jnp.int32),
            pltpu.VMEM((CHUNK, PW), jnp.float32),
            pltpu.SemaphoreType.DMA,
        ],
    )
    def k(idx_hbm, tbl_hbm, emb_hbm, idx_v, row_v, rows_v, sem):
        wid = lax.axis_index("s") * NC + lax.axis_index("c")
        base = wid * bpw
        pltpu.sync_copy(idx_hbm.at[pl.ds(base, bpw)], idx_v)

        def chunk_body(c, _):
            c0 = c * CHUNK
            for v in range(CHUNK // L):
                row_v[pl.ds(v * L, L)] = idx_v[pl.ds(c0 + v * L, L)] >> 1
            pltpu.async_copy(tbl_hbm.at[row_v], rows_v, sem).wait()
            pltpu.sync_copy(rows_v, emb_hbm.at[pl.ds(base + c0, CHUNK)])
            return _

        lax.fori_loop(0, n_chunks, chunk_body, None)

    return k


def _make_dot(batch):
    bpw = batch // NW
    n_chunks = bpw // CHUNK
    mesh = plsc.VectorSubcoreMesh(core_axis_name="c", subcore_axis_name="s", num_cores=NC)

    @functools.partial(
        pl.kernel,
        mesh=mesh,
        out_type=jax.ShapeDtypeStruct((batch,), jnp.float32),
        compiler_params=pltpu.CompilerParams(
            use_tc_tiling_on_sc=True, skip_device_barrier=True),
        scratch_types=[
            pltpu.VMEM((bpw,), jnp.int32),
            pltpu.VMEM((bpw,), jnp.int32),
            pltpu.VMEM((CHUNK, PW), jnp.float32),
            pltpu.VMEM((CHUNK, PW), jnp.float32),
            pltpu.VMEM((bpw,), jnp.float32),
        ],
    )
    def k(user_hbm, item_hbm, uemb_hbm, iemb_hbm, out_hbm,
          uidx_v, iidx_v, urows_v, irows_v, out_v):
        wid = lax.axis_index("s") * NC + lax.axis_index("c")
        base = wid * bpw

        pltpu.sync_copy(user_hbm.at[pl.ds(base, bpw)], uidx_v)
        pltpu.sync_copy(item_hbm.at[pl.ds(base, bpw)], iidx_v)

        lanes = lax.iota(jnp.int32, L)

        def perm_xor(v, s):
            return v.at[lanes ^ s].get(mode="promise_in_bounds")

        def chunk_body(c, _):
            c0 = c * CHUNK
            pltpu.sync_copy(uemb_hbm.at[pl.ds(base + c0, CHUNK)], urows_v)
            pltpu.sync_copy(iemb_hbm.at[pl.ds(base + c0, CHUNK)], irows_v)

            def group(g, _):
                r0 = g * L
                usel = (uidx_v[pl.ds(c0 + r0, L)] & 1) * D
                isel = (iidx_v[pl.ds(c0 + r0, L)] & 1) * D
                accs = []
                for r in range(L):
                    uo = usel[r]
                    io = isel[r]
                    acc = (urows_v[r0 + r, pl.ds(uo, L)] *
                           irows_v[r0 + r, pl.ds(io, L)])
                    for jc in range(1, D // L):
                        acc = acc + (
                            urows_v[r0 + r, pl.ds(uo + jc * L, L)] *
                            irows_v[r0 + r, pl.ds(io + jc * L, L)])
                    accs.append(acc)
                s = 1
                while len(accs) > 1:
                    lo_mask = (lanes & s) == 0
                    nxt = []
                    for i in range(0, len(accs), 2):
                        a, b = accs[i], accs[i + 1]
                        merged = (jnp.where(lo_mask, a, perm_xor(b, s)) +
                                  jnp.where(lo_mask, perm_xor(a, s), b))
                        nxt.append(merged)
                    accs = nxt
                    s *= 2
                out_v[pl.ds(c0 + r0, L)] = accs[0]
                return _

            lax.fori_loop(0, CHUNK // L, group, None)
            return _

        lax.fori_loop(0, n_chunks, chunk_body, None)

        pltpu.sync_copy(out_v, out_hbm.at[pl.ds(base, bpw)])

    return k


def kernel(user, item, user_table, item_table):
    batch = user.shape[0]
    n_rows, dim = user_table.shape
    gather = _make_gather(batch)
    ut2 = user_table.reshape(n_rows * dim // PW, PW)
    it2 = item_table.reshape(n_rows * dim // PW, PW)
    uemb = gather(user, ut2)
    iemb = gather(item, it2)
    return _make_dot(batch)(user, item, uemb, iemb)
